# Initial kernel scaffold; baseline (speedup 1.0000x reference)
import functools

import jax
import jax.numpy as jnp
from jax import lax
from jax.experimental import pallas as pl
from jax.experimental.pallas import tpu as pltpu
from jax.experimental.pallas import tpu_sc as plsc

_NC = 2
_NS = 16
_NW = _NC * _NS
_L = 16

_B = 16384
_NEG = 20
_D = 64
_BW = _B // _NW
_CB = 32
_NCHUNK = _BW // _CB
_NROWS = 1 + _NEG


def _sc_scores_kernel(tt_hbm, ct_hbm, tidx_hbm, cidx_hbm, nidx_hbm,
                      scores_hbm,
                      tidx_v, cidx_v, nidx_v, tbuf, cbuf, nbuf, scores_v,
                      sem0, sem1):
    wid = lax.axis_index("s") * _NC + lax.axis_index("c")
    base = wid * _BW

    pltpu.sync_copy(tidx_hbm.at[pl.ds(base, _BW)], tidx_v)
    pltpu.sync_copy(cidx_hbm.at[pl.ds(base, _BW)], cidx_v)
    pltpu.sync_copy(nidx_hbm.at[pl.ds(base * _NEG, _BW * _NEG)], nidx_v)

    sems = [sem0, sem1]

    def issue(g):
        b = g % 2
        ht = pltpu.async_copy(
            tt_hbm.at[tidx_v.at[pl.ds(g * _CB, _CB)]], tbuf.at[b], sems[b])
        hc = pltpu.async_copy(
            ct_hbm.at[cidx_v.at[pl.ds(g * _CB, _CB)]], cbuf.at[b], sems[b])
        hn = pltpu.async_copy(
            ct_hbm.at[nidx_v.at[pl.ds(g * _CB * _NEG, _CB * _NEG)]],
            nbuf.at[b], sems[b])
        return (ht, hc, hn)

    lanes = lax.iota(jnp.int32, _L)

    def compute(g):
        b = g % 2
        tb, cb, nb = tbuf.at[b], cbuf.at[b], nbuf.at[b]
        for grp in range(_CB // _L):
            erow = lanes + (grp * _L)
            nrowbase = erow * _NEG

            def body(d, accs):
                dcol = jnp.full((_L,), d, dtype=jnp.int32)
                t_d = plsc.load_gather(tb, [erow, dcol])
                c_d = plsc.load_gather(cb, [erow, dcol])
                out = [accs[0] + t_d * c_d]
                for k in range(_NEG):
                    n_d = plsc.load_gather(nb, [nrowbase + k, dcol])
                    out.append(accs[k + 1] + n_d * t_d)
                return tuple(out)

            accs = lax.fori_loop(
                0, _D, body,
                tuple(jnp.zeros((_L,), jnp.float32) for _ in range(_NROWS)))
            sl = pl.ds(g * _CB + grp * _L, _L)
            scores_v[0, sl] = -accs[0]
            for k in range(_NEG):
                scores_v[1 + k, sl] = accs[k + 1]

    handles = issue(0)
    for g in range(_NCHUNK):
        nxt = issue(g + 1) if g + 1 < _NCHUNK else None
        for h in handles:
            h.wait()
        compute(g)
        handles = nxt

    pltpu.sync_copy(scores_v, scores_hbm.at[wid])


def _sc_scores(target_idx, context_idx, neg_idx_flat, target_table,
               context_table):
    mesh = plsc.VectorSubcoreMesh(core_axis_name="c", subcore_axis_name="s")
    kern = functools.partial(
        pl.kernel,
        mesh=mesh,
        out_type=jax.ShapeDtypeStruct((_NW, _NROWS, _BW), jnp.float32),
        scratch_types=[
            pltpu.VMEM((_BW,), jnp.int32),
            pltpu.VMEM((_BW,), jnp.int32),
            pltpu.VMEM((_BW * _NEG,), jnp.int32),
            pltpu.VMEM((2, _CB, _D), jnp.float32),
            pltpu.VMEM((2, _CB, _D), jnp.float32),
            pltpu.VMEM((2, _CB * _NEG, _D), jnp.float32),
            pltpu.VMEM((_NROWS, _BW), jnp.float32),
            pltpu.SemaphoreType.DMA,
            pltpu.SemaphoreType.DMA,
        ],
    )(_sc_scores_kernel)
    return kern(target_table, context_table, target_idx, context_idx,
                neg_idx_flat)


def _tc_loss_kernel(scores_ref, out_ref):
    x = scores_ref[...]
    sp = jnp.maximum(x, 0.0) + jnp.log1p(jnp.exp(-jnp.abs(x)))
    out_ref[...] = jnp.full((1, 1), jnp.sum(sp) * (1.0 / _B), jnp.float32)


def _tc_loss(scores2d):
    out = pl.pallas_call(
        _tc_loss_kernel,
        out_shape=jax.ShapeDtypeStruct((1, 1), jnp.float32),
    )(scores2d)
    return out[0, 0]


def kernel(target_idx, context_idx, neg_idx, target_table, context_table):
    target_idx = target_idx.astype(jnp.int32)
    context_idx = context_idx.astype(jnp.int32)
    neg_idx_flat = neg_idx.astype(jnp.int32).reshape(_B * _NEG)
    scores = _sc_scores(target_idx, context_idx, neg_idx_flat,
                        target_table, context_table)
    return _tc_loss(scores.reshape(_NW * _NROWS, _BW))


# speedup vs baseline: 5.3273x; 5.3273x over previous
import functools

import jax
import jax.numpy as jnp
from jax import lax
from jax.experimental import pallas as pl
from jax.experimental.pallas import tpu as pltpu
from jax.experimental.pallas import tpu_sc as plsc

_NC = 2
_NS = 16
_NW = _NC * _NS
_L = 16

_B = 16384
_NEG = 20
_D = 64
_DV = _D // _L
_BW = _B // _NW
_CB = 32
_NCHUNK = _BW // _CB
_NROWS = 1 + _NEG


def _sc_scores_kernel(tt_hbm, ct_hbm, tidx_hbm, cidx_hbm, nidx_hbm,
                      scores_hbm,
                      tidx_v, cidx_v, nidx_v, tbuf, cbuf, nbuf, scores_v,
                      sem):
    wid = lax.axis_index("s") * _NC + lax.axis_index("c")
    base = wid * _BW

    pltpu.sync_copy(tidx_hbm.at[pl.ds(base, _BW)], tidx_v)
    pltpu.sync_copy(cidx_hbm.at[pl.ds(base, _BW)], cidx_v)
    pltpu.sync_copy(nidx_hbm.at[pl.ds(base * _NEG, _BW * _NEG)], nidx_v)

    lanes = lax.iota(jnp.int32, _L)

    def chunk_body(g, carry):
        ht = pltpu.async_copy(
            tt_hbm.at[tidx_v.at[pl.ds(g * _CB, _CB)]], tbuf, sem)
        hc = pltpu.async_copy(
            ct_hbm.at[cidx_v.at[pl.ds(g * _CB, _CB)]], cbuf, sem)
        hn = pltpu.async_copy(
            ct_hbm.at[nidx_v.at[pl.ds(g * _CB * _NEG, _CB * _NEG)]],
            nbuf, sem)
        ht.wait()
        hc.wait()
        hn.wait()

        def group_body(grp, carry2):
            def elem_body(i, accs):
                e = grp * _L + i
                sel = lanes == i
                ts = [tbuf[e, pl.ds(j * _L, _L)] for j in range(_DV)]
                cs = [cbuf[e, pl.ds(j * _L, _L)] for j in range(_DV)]
                p = ts[0] * cs[0]
                for j in range(1, _DV):
                    p = p + ts[j] * cs[j]
                out = [jnp.where(sel, -jnp.sum(p), accs[0])]
                nrow = e * _NEG
                for k in range(_NEG):
                    q = ts[0] * nbuf[nrow + k, pl.ds(0, _L)]
                    for j in range(1, _DV):
                        q = q + ts[j] * nbuf[nrow + k, pl.ds(j * _L, _L)]
                    out.append(jnp.where(sel, jnp.sum(q), accs[1 + k]))
                return tuple(out)

            accs = lax.fori_loop(
                0, _L, elem_body,
                tuple(jnp.zeros((_L,), jnp.float32) for _ in range(_NROWS)))
            col = pl.ds(g * _CB + grp * _L, _L)
            for r in range(_NROWS):
                scores_v[r, col] = accs[r]
            return carry2

        return lax.fori_loop(0, _CB // _L, group_body, carry)

    lax.fori_loop(0, _NCHUNK, chunk_body, 0)

    pltpu.sync_copy(scores_v, scores_hbm.at[wid])


def _sc_scores(target_idx, context_idx, neg_idx_flat, target_table,
               context_table):
    mesh = plsc.VectorSubcoreMesh(core_axis_name="c", subcore_axis_name="s")
    kern = functools.partial(
        pl.kernel,
        mesh=mesh,
        compiler_params=pltpu.CompilerParams(needs_layout_passes=False,
                                             use_tc_tiling_on_sc=False),
        out_type=jax.ShapeDtypeStruct((_NW, _NROWS, _BW), jnp.float32),
        scratch_types=[
            pltpu.VMEM((_BW,), jnp.int32),
            pltpu.VMEM((_BW,), jnp.int32),
            pltpu.VMEM((_BW * _NEG,), jnp.int32),
            pltpu.VMEM((_CB, _D), jnp.float32),
            pltpu.VMEM((_CB, _D), jnp.float32),
            pltpu.VMEM((_CB * _NEG, _D), jnp.float32),
            pltpu.VMEM((_NROWS, _BW), jnp.float32),
            pltpu.SemaphoreType.DMA,
        ],
    )(_sc_scores_kernel)
    return kern(target_table, context_table, target_idx, context_idx,
                neg_idx_flat)


def _tc_loss_kernel(scores_ref, out_ref):
    x = scores_ref[...]
    sp = jnp.maximum(x, 0.0) + jnp.log1p(jnp.exp(-jnp.abs(x)))
    out_ref[...] = jnp.full((1, 1), jnp.sum(sp) * (1.0 / _B), jnp.float32)


def _tc_loss(scores2d):
    out = pl.pallas_call(
        _tc_loss_kernel,
        out_shape=jax.ShapeDtypeStruct((1, 1), jnp.float32),
    )(scores2d)
    return out[0, 0]


def kernel(target_idx, context_idx, neg_idx, target_table, context_table):
    target_idx = target_idx.astype(jnp.int32)
    context_idx = context_idx.astype(jnp.int32)
    neg_idx_flat = neg_idx.astype(jnp.int32).reshape(_B * _NEG)
    scores = _sc_scores(target_idx, context_idx, neg_idx_flat,
                        target_table, context_table)
    return _tc_loss(scores.reshape(_NW * _NROWS, _BW))


# speedup vs baseline: 7.0096x; 1.3158x over previous
import functools

import jax
import jax.numpy as jnp
from jax import lax
from jax.experimental import pallas as pl
from jax.experimental.pallas import tpu as pltpu
from jax.experimental.pallas import tpu_sc as plsc

_NC = 2
_NS = 16
_NW = _NC * _NS
_L = 16

_B = 16384
_NEG = 20
_D = 64
_DV = _D // _L
_BW = _B // _NW
_CB = 16
_NCHUNK = _BW // _CB
_SBLK = 8
_NROWS = 1 + _NEG


def _sc_scores_kernel(tt_hbm, ct_hbm, tidx_hbm, cidx_hbm, nidx_hbm,
                      scores_hbm,
                      tidx_v, cidx_v, nidx_v,
                      tbufA, cbufA, nbufA, tbufB, cbufB, nbufB,
                      scores_v, semA, semB):
    wid = lax.axis_index("s") * _NC + lax.axis_index("c")
    base = wid * _BW

    pltpu.sync_copy(tidx_hbm.at[pl.ds(base, _BW)], tidx_v.at[pl.ds(0, _BW)])
    pltpu.sync_copy(cidx_hbm.at[pl.ds(base, _BW)], cidx_v.at[pl.ds(0, _BW)])
    pltpu.sync_copy(nidx_hbm.at[pl.ds(base * _NEG, _BW * _NEG)],
                    nidx_v.at[pl.ds(0, _BW * _NEG)])

    bufs = [(tbufA, cbufA, nbufA, semA), (tbufB, cbufB, nbufB, semB)]

    def issue(g, b):
        tb, cb, nb, sem = bufs[b]

        def elem(e, carry):
            col = g * _CB + e
            ti = tidx_v[pl.ds(col, _L)][0]
            ci = cidx_v[pl.ds(col, _L)][0]
            pltpu.async_copy(tt_hbm.at[ti], tb.at[e], sem)
            pltpu.async_copy(ct_hbm.at[ci], cb.at[e], sem)
            ncol = col * _NEG
            nrow = e * _NEG

            def negk(k4, carry2):
                for j in range(4):
                    ni = nidx_v[pl.ds(ncol + k4 * 4 + j, _L)][0]
                    pltpu.async_copy(ct_hbm.at[ni],
                                     nb.at[nrow + k4 * 4 + j], sem)
                return carry2

            lax.fori_loop(0, _NEG // 4, negk, 0)
            return carry

        lax.fori_loop(0, _CB, elem, 0)

    def drain(b):
        tb, cb, nb, sem = bufs[b]
        pltpu.make_async_copy(tt_hbm.at[pl.ds(0, _CB)], tb, sem).wait()
        pltpu.make_async_copy(tt_hbm.at[pl.ds(0, _CB)], cb, sem).wait()
        pltpu.make_async_copy(ct_hbm.at[pl.ds(0, _CB * _NEG)], nb,
                              sem).wait()

    lanes = lax.iota(jnp.int32, _L)

    def compute(g, b):
        tb, cb, nb, _ = bufs[b]

        def elem_body(i, accs):
            sel = lanes == i
            ts = [tb[i, pl.ds(j * _L, _L)] for j in range(_DV)]
            cs = [cb[i, pl.ds(j * _L, _L)] for j in range(_DV)]
            p = ts[0] * cs[0]
            for j in range(1, _DV):
                p = p + ts[j] * cs[j]
            out = [jnp.where(sel, -jnp.sum(p), accs[0])]
            nrow = i * _NEG
            for k in range(_NEG):
                q = ts[0] * nb[nrow + k, pl.ds(0, _L)]
                for j in range(1, _DV):
                    q = q + ts[j] * nb[nrow + k, pl.ds(j * _L, _L)]
                out.append(jnp.where(sel, jnp.sum(q), accs[1 + k]))
            return tuple(out)

        accs = lax.fori_loop(
            0, _L, elem_body,
            tuple(jnp.zeros((_L,), jnp.float32) for _ in range(_NROWS)))
        col = pl.ds((g % _SBLK) * _CB, _L)
        for r in range(_NROWS):
            scores_v[r, col] = accs[r]

    def flush(g):
        blk = (g // _SBLK) * (_SBLK * _CB)
        pltpu.sync_copy(scores_v,
                        scores_hbm.at[:, pl.ds(base + blk, _SBLK * _CB)])

    def pair_body(p, carry):
        c0 = p * 2
        c1 = c0 + 1
        issue(c1, 1)
        drain(0)
        compute(c0, 0)

        @pl.when(p < _NCHUNK // 2 - 1)
        def _():
            issue(c1 + 1, 0)

        drain(1)
        compute(c1, 1)

        @pl.when(c1 % _SBLK == _SBLK - 1)
        def _():
            flush(c1)

        return carry

    issue(0, 0)
    lax.fori_loop(0, _NCHUNK // 2, pair_body, 0)


def _sc_scores(target_idx, context_idx, neg_idx_flat, target_table,
               context_table):
    mesh = plsc.VectorSubcoreMesh(core_axis_name="c", subcore_axis_name="s")
    kern = functools.partial(
        pl.kernel,
        mesh=mesh,
        compiler_params=pltpu.CompilerParams(needs_layout_passes=False),
        out_type=jax.ShapeDtypeStruct((_NROWS, _B), jnp.float32),
        scratch_types=[
            pltpu.VMEM((_BW + _L,), jnp.int32),
            pltpu.VMEM((_BW + _L,), jnp.int32),
            pltpu.VMEM((_BW * _NEG + _L,), jnp.int32),
            pltpu.VMEM((_CB, _D), jnp.float32),
            pltpu.VMEM((_CB, _D), jnp.float32),
            pltpu.VMEM((_CB * _NEG, _D), jnp.float32),
            pltpu.VMEM((_CB, _D), jnp.float32),
            pltpu.VMEM((_CB, _D), jnp.float32),
            pltpu.VMEM((_CB * _NEG, _D), jnp.float32),
            pltpu.VMEM((_NROWS, _SBLK * _CB), jnp.float32),
            pltpu.SemaphoreType.DMA,
            pltpu.SemaphoreType.DMA,
        ],
    )(_sc_scores_kernel)
    return kern(target_table, context_table, target_idx, context_idx,
                neg_idx_flat)


def _tc_loss_kernel(scores_ref, out_ref):
    x = scores_ref[...]
    sp = jnp.maximum(x, 0.0) + jnp.log1p(jnp.exp(-jnp.abs(x)))
    out_ref[...] = jnp.full((1, 1), jnp.sum(sp) * (1.0 / _B), jnp.float32)


def _tc_loss(scores2d):
    out = pl.pallas_call(
        _tc_loss_kernel,
        out_shape=jax.ShapeDtypeStruct((1, 1), jnp.float32),
    )(scores2d)
    return out[0, 0]


def kernel(target_idx, context_idx, neg_idx, target_table, context_table):
    target_idx = target_idx.astype(jnp.int32)
    context_idx = context_idx.astype(jnp.int32)
    neg_idx_flat = neg_idx.astype(jnp.int32).reshape(_B * _NEG)
    scores = _sc_scores(target_idx, context_idx, neg_idx_flat,
                        target_table, context_table)
    return _tc_loss(scores)
